# baseline (speedup 1.0000x reference)
import jax
import jax.numpy as jnp
from jax import lax
from jax.experimental import pallas as pl
from jax.experimental.pallas import tpu as pltpu, tpu_sc as plsc
import functools

_MAX_LEN = 8192
_NC = 2
_NS = 16
_NW = _NC * _NS
_L = 16

_BATCH = 4
_SEQ = 8192
_D = 1024
_R = 32
_SEQ_PER_W = _SEQ // _NW
_CHUNKS = _SEQ_PER_W // _R
_VPR = _D // _L


def _sc_body(x_hbm, emb_hbm, out_hbm, emb_v, xbuf):
    c = lax.axis_index("c")
    s = lax.axis_index("s")
    wid = s * _NC + c

    def add_row(r, _):
        def add_vreg(j, _):
            sl = pl.ds(j * _L, _L)
            plsc.addupdate(xbuf.at[r, sl], emb_v[r, sl])
            return _
        return lax.fori_loop(0, _VPR, add_vreg, _)

    for chunk in range(_CHUNKS):
        s0 = wid * _SEQ_PER_W + chunk * _R
        pltpu.sync_copy(emb_hbm.at[pl.ds(_MAX_LEN + s0, _R)], emb_v)
        for b in range(_BATCH):
            pltpu.sync_copy(x_hbm.at[b, pl.ds(s0, _R)], xbuf)
            lax.fori_loop(0, _R, add_row, 0)
            pltpu.sync_copy(xbuf, out_hbm.at[b, pl.ds(s0, _R)])


@functools.partial(
    pl.kernel,
    out_type=jax.ShapeDtypeStruct((_BATCH, _SEQ, _D), jnp.float32),
    mesh=plsc.VectorSubcoreMesh(core_axis_name="c", subcore_axis_name="s"),
    scratch_types=[
        pltpu.VMEM((_R, _D), jnp.float32),
        pltpu.VMEM((_R, _D), jnp.float32),
    ],
)
def _sc_kernel(x_hbm, emb_hbm, out_hbm, emb_v, xbuf):
    _sc_body(x_hbm, emb_hbm, out_hbm, emb_v, xbuf)


def kernel(x, rel_pos_emb):
    return _sc_kernel(x, rel_pos_emb)


# speedup vs baseline: 4.8737x; 4.8737x over previous
import jax
import jax.numpy as jnp
from jax.experimental import pallas as pl

_MAX_LEN = 8192
_S_BLK = 1024
_B_BLK = 2


def _add_body(x_ref, emb_ref, out_ref):
    out_ref[...] = x_ref[...] + emb_ref[...][None, :, :]


def kernel(x, rel_pos_emb):
    batch, seq_len, d_model = x.shape
    n_blocks = seq_len // _S_BLK
    emb_off = _MAX_LEN // _S_BLK
    return pl.pallas_call(
        _add_body,
        grid=(n_blocks, batch // _B_BLK),
        in_specs=[
            pl.BlockSpec((_B_BLK, _S_BLK, d_model), lambda j, b: (b, j, 0)),
            pl.BlockSpec((_S_BLK, d_model), lambda j, b: (emb_off + j, 0)),
        ],
        out_specs=pl.BlockSpec((_B_BLK, _S_BLK, d_model), lambda j, b: (b, j, 0)),
        out_shape=jax.ShapeDtypeStruct((batch, seq_len, d_model), x.dtype),
    )(x, rel_pos_emb)
